# baseline (speedup 1.0000x reference)
import jax
import jax.numpy as jnp
from jax import lax
from jax.experimental import pallas as pl
from jax.experimental.pallas import tpu as pltpu

_PAR = pltpu.CompilerParams(dimension_semantics=("parallel",))

_CFGS = [
    (48, 24, 3, 1, 1),
    (24, 32, 3, 2, 6),
    (32, 32, 3, 1, 6),
    (32, 56, 5, 2, 6),
    (56, 112, 3, 2, 6),
    (112, 160, 5, 1, 6),
    (160, 272, 5, 2, 6),
    (272, 448, 3, 1, 6),
]


def _same_pad(size, k, s):
    out = -(-size // s)
    pad = max((out - 1) * s + k - size, 0)
    return pad // 2, pad - pad // 2, out


def _full_spec(a):
    nd = a.ndim
    return pl.BlockSpec(a.shape, lambda n, _nd=nd: (0,) * _nd)


def _s2d(x, H, W):
    N, _, C = x.shape
    y = x.reshape(N, H // 2, 2, W // 2, 2, C)
    y = jnp.transpose(y, (0, 1, 3, 2, 4, 5))
    return y.reshape(N, H // 2, W // 2, 4 * C)


def _mbconv(x, H, W, cfg, p):
    in_ch, out_ch, k, s, expand = cfg
    N = x.shape[0]
    mid = in_ch * expand
    pt, pb, Ho = _same_pad(H, k, s)
    pL, pR, Wo = _same_pad(W, k, s)
    use_res = (s == 1 and in_ch == out_ch)
    has_exp = expand != 1

    if s == 2:
        H2, W2 = H // 2, W // 2
        TA = max(0, -min((kh - pt) // 2 for kh in range(k)))
        BA = max(0, max((kh - pt) // 2 for kh in range(k)) + Ho - H2)
        LA = max(0, -min((kw - pL) // 2 for kw in range(k)))
        RA = max(0, max((kw - pL) // 2 for kw in range(k)) + Wo - W2)
        x_in = x
        in_spec = pl.BlockSpec((1, H2, W2, 4 * in_ch),
                               lambda n: (n, 0, 0, 0))
        midp = -(-mid // 128) * 128
        ew_arg = jnp.kron(jnp.eye(4, dtype=p["exp_w"].dtype),
                          jnp.pad(p["exp_w"], ((0, 0), (0, midp - mid))))
        eb_arg = jnp.tile(jnp.pad(p["exp_b"], ((0, 0), (0, midp - mid))),
                          (1, 4))
    else:
        x_in = x
        in_spec = pl.BlockSpec((1, H * W, in_ch), lambda n: (n, 0, 0))
        if has_exp:
            ew_arg, eb_arg = p["exp_w"], p["exp_b"]

    def body(*refs):
        if has_exp:
            (x_ref, ew, eb, dw, db, s1, c1, s2, c2, pw, pbias, o_ref) = refs
        else:
            (x_ref, dw, db, s1, c1, s2, c2, pw, pbias, o_ref) = refs

        def expand_fn(r):
            if not has_exp:
                return r
            h = jnp.dot(r, ew[...], preferred_element_type=jnp.float32)
            h = h + eb[...]
            return (h * jax.nn.sigmoid(h)).astype(jnp.bfloat16)

        if s == 1:
            x2 = x_ref[0]
            h = expand_fn(x2)
            hp = jnp.pad(h.reshape(H, W, mid),
                         ((pt, pb), (pL, pR), (0, 0)))
            taps = {(0, 0): hp}
        else:
            blk = x_ref[0]
            e = expand_fn(blk.reshape(H2 * W2, 4 * in_ch))
            e3 = e.reshape(H2, W2, 4 * midp)
            taps = {}
            for a in range(2):
                for b2 in range(2):
                    g = (a * 2 + b2) * midp
                    pl_ = e3[:, :, g:g + mid]
                    taps[(a, b2)] = jnp.pad(
                        pl_, ((TA, BA), (LA, RA), (0, 0)))
        acc = jnp.zeros((Ho, Wo, mid), jnp.float32)
        for kh in range(k):
            for kw in range(k):
                if s == 1:
                    src = taps[(0, 0)]
                    ah, aw = kh, kw
                else:
                    src = taps[((kh - pt) % 2, (kw - pL) % 2)]
                    ah = (kh - pt) // 2 + TA
                    aw = (kw - pL) // 2 + LA
                tap = src[ah:ah + Ho, aw:aw + Wo, :]
                acc = acc + tap.astype(jnp.float32) * dw[kh * k + kw]
        y = acc + db[...]
        y = y * jax.nn.sigmoid(y)
        y2 = y.reshape(Ho * Wo, mid)
        pooled = jnp.mean(y2, axis=0, keepdims=True)
        t = jnp.dot(pooled.astype(jnp.bfloat16), s1[...],
                    preferred_element_type=jnp.float32) + c1[...]
        t = t * jax.nn.sigmoid(t)
        g = jnp.dot(t.astype(jnp.bfloat16), s2[...],
                    preferred_element_type=jnp.float32) + c2[...]
        gate = jax.nn.sigmoid(g)
        ys = (y2 * gate).astype(jnp.bfloat16)
        out = jnp.dot(ys, pw[...],
                      preferred_element_type=jnp.float32) + pbias[...]
        if use_res:
            out = out + x2.astype(jnp.float32)
        o_ref[0] = out.astype(jnp.bfloat16)

    names = ["dw_w", "dw_b", "se_w1", "se_b1", "se_w2", "se_b2",
             "proj_w", "proj_b"]
    args = [x_in] + ([ew_arg, eb_arg] if has_exp else []) \
        + [p[nm] for nm in names]
    specs = [in_spec]
    specs += [_full_spec(a) for a in args[1:]]
    out = pl.pallas_call(
        body,
        out_shape=jax.ShapeDtypeStruct((N, Ho * Wo, out_ch), jnp.bfloat16),
        grid=(N,),
        in_specs=specs,
        out_specs=pl.BlockSpec((1, Ho * Wo, out_ch), lambda n: (n, 0, 0)),
        compiler_params=_PAR,
    )(*args)
    return out, Ho, Wo


def _stem_b0(x, w, b, p):
    N = x.shape[0]
    C = 48
    xs = x.astype(jnp.bfloat16).reshape(N, 3, 56, 4, 56, 4)
    xs = jnp.transpose(xs, (0, 2, 4, 3, 5, 1)).reshape(N, 56, 56, 48)
    pw4 = jnp.kron(jnp.eye(4, dtype=p["proj_w"].dtype),
                   jnp.pad(p["proj_w"], ((0, 128 - C), (0, 0))))
    pb4 = jnp.tile(p["proj_b"], (1, 4))

    def body(x_ref, w_ref, b_ref, dw, db, s1, c1, s2, c2, pwr, pbr, o_ref):
        src = jnp.pad(x_ref[0], ((0, 1), (0, 1), (0, 0)))
        planes = {}
        for u in range(2):
            for v in range(2):
                pieces = []
                for kh in range(3):
                    for kw in range(3):
                        ish, a = (2 * u + kh) // 4, (2 * u + kh) % 4
                        jsh, bb = (2 * v + kw) // 4, (2 * v + kw) % 4
                        lb = (a * 4 + bb) * 3
                        pieces.append(src[ish:ish + 56, jsh:jsh + 56,
                                          lb:lb + 3])
                pat = jnp.concatenate(pieces, axis=-1).reshape(56 * 56, 27)
                yv = jnp.dot(pat, w_ref[...],
                             preferred_element_type=jnp.float32) + b_ref[...]
                yv = yv * jax.nn.sigmoid(yv)
                st = yv.astype(jnp.bfloat16).reshape(56, 56, 48)
                planes[(u, v)] = jnp.pad(st, ((1, 1), (1, 1), (0, 0)))
        ys = []
        pooled = None
        for u in range(2):
            for v in range(2):
                acc = jnp.zeros((56, 56, C), jnp.float32)
                for kh in range(3):
                    for kw in range(3):
                        ra, d = (u + kh - 1) % 2, (u + kh - 1) // 2 + 1
                        ca, e = (v + kw - 1) % 2, (v + kw - 1) // 2 + 1
                        tap = planes[(ra, ca)][d:d + 56, e:e + 56, :]
                        acc = acc + tap.astype(jnp.float32) * dw[kh * 3 + kw]
                yv = acc + db[...]
                yv = yv * jax.nn.sigmoid(yv)
                ys.append(yv)
                pm = jnp.mean(yv.reshape(56 * 56, C), axis=0, keepdims=True)
                pooled = pm if pooled is None else pooled + pm
        pooled = pooled * 0.25
        t = jnp.dot(pooled.astype(jnp.bfloat16), s1[...],
                    preferred_element_type=jnp.float32) + c1[...]
        t = t * jax.nn.sigmoid(t)
        g = jnp.dot(t.astype(jnp.bfloat16), s2[...],
                    preferred_element_type=jnp.float32) + c2[...]
        gate = jax.nn.sigmoid(g)
        gated = [jnp.pad((y.reshape(56 * 56, C) * gate).astype(jnp.bfloat16),
                         ((0, 0), (0, 128 - C)))
                 for y in ys]
        big = jnp.concatenate(gated, axis=-1)
        out = jnp.dot(big, pwr[...],
                      preferred_element_type=jnp.float32) + pbr[...]
        o_ref[0] = out.astype(jnp.bfloat16).reshape(56, 56, 96)

    wargs = [w, b, p["dw_w"], p["dw_b"], p["se_w1"], p["se_b1"],
             p["se_w2"], p["se_b2"], pw4, pb4]
    return pl.pallas_call(
        body,
        out_shape=jax.ShapeDtypeStruct((N, 56, 56, 96), jnp.bfloat16),
        grid=(N,),
        in_specs=[pl.BlockSpec((1, 56, 56, 48), lambda n: (n, 0, 0, 0))]
        + [_full_spec(a) for a in wargs],
        out_specs=pl.BlockSpec((1, 56, 56, 96), lambda n: (n, 0, 0, 0)),
        compiler_params=_PAR,
    )(xs, *wargs)


def _b1_b2(x, p1, p2):
    N = x.shape[0]
    m1, m2 = 144, 192
    m1p = 256
    ew4 = jnp.kron(jnp.eye(4, dtype=p1["exp_w"].dtype),
                   jnp.pad(p1["exp_w"], ((0, 0), (0, m1p - m1))))
    eb4 = jnp.tile(jnp.pad(p1["exp_b"], ((0, 0), (0, m1p - m1))), (1, 4))
    wargs = [ew4, eb4, p1["dw_w"], p1["dw_b"], p1["se_w1"], p1["se_b1"],
             p1["se_w2"], p1["se_b2"], p1["proj_w"], p1["proj_b"],
             p2["exp_w"], p2["exp_b"], p2["dw_w"], p2["dw_b"], p2["se_w1"],
             p2["se_b1"], p2["se_w2"], p2["se_b2"], p2["proj_w"],
             p2["proj_b"]]

    def body(x_ref, e1w, e1b, d1w, d1b, s11, c11, s12, c12, p1w, p1b,
             e2w, e2b, d2w, d2b, s21, c21, s22, c22, p2w, p2b, o_ref):
        blk = x_ref[0]
        e = jnp.dot(blk.reshape(56 * 56, 96), e1w[...],
                    preferred_element_type=jnp.float32) + e1b[...]
        e = (e * jax.nn.sigmoid(e)).astype(jnp.bfloat16)
        e = e.reshape(56, 56, 4 * m1p)
        taps = {}
        for a in range(2):
            for b2_ in range(2):
                g = (a * 2 + b2_) * m1p
                pl_ = e[:, :, g:g + m1]
                taps[(a, b2_)] = jnp.pad(pl_, ((0, 1), (0, 1), (0, 0)))
        acc = jnp.zeros((56, 56, m1), jnp.float32)
        for kh in range(3):
            for kw in range(3):
                src = taps[(kh % 2, kw % 2)]
                tap = src[kh // 2:kh // 2 + 56, kw // 2:kw // 2 + 56, :]
                acc = acc + tap.astype(jnp.float32) * d1w[kh * 3 + kw]
        y = acc + d1b[...]
        y = y * jax.nn.sigmoid(y)
        y2 = y.reshape(3136, m1)
        pooled = jnp.mean(y2, axis=0, keepdims=True)
        t = jnp.dot(pooled.astype(jnp.bfloat16), s11[...],
                    preferred_element_type=jnp.float32) + c11[...]
        t = t * jax.nn.sigmoid(t)
        g = jnp.dot(t.astype(jnp.bfloat16), s12[...],
                    preferred_element_type=jnp.float32) + c12[...]
        gate = jax.nn.sigmoid(g)
        ys = (y2 * gate).astype(jnp.bfloat16)
        h1 = jnp.dot(ys, p1w[...],
                     preferred_element_type=jnp.float32) + p1b[...]
        h1 = h1.astype(jnp.bfloat16)
        e2 = jnp.dot(h1, e2w[...],
                     preferred_element_type=jnp.float32) + e2b[...]
        e2 = (e2 * jax.nn.sigmoid(e2)).astype(jnp.bfloat16)
        hp = jnp.pad(e2.reshape(56, 56, m2), ((1, 1), (1, 1), (0, 0)))
        acc2 = jnp.zeros((56, 56, m2), jnp.float32)
        for kh in range(3):
            for kw in range(3):
                tap = hp[kh:kh + 56, kw:kw + 56, :]
                acc2 = acc2 + tap.astype(jnp.float32) * d2w[kh * 3 + kw]
        y = acc2 + d2b[...]
        y = y * jax.nn.sigmoid(y)
        y2 = y.reshape(3136, m2)
        pooled = jnp.mean(y2, axis=0, keepdims=True)
        t = jnp.dot(pooled.astype(jnp.bfloat16), s21[...],
                    preferred_element_type=jnp.float32) + c21[...]
        t = t * jax.nn.sigmoid(t)
        g = jnp.dot(t.astype(jnp.bfloat16), s22[...],
                    preferred_element_type=jnp.float32) + c22[...]
        gate = jax.nn.sigmoid(g)
        ys = (y2 * gate).astype(jnp.bfloat16)
        out = jnp.dot(ys, p2w[...],
                      preferred_element_type=jnp.float32) + p2b[...]
        out = out + h1.astype(jnp.float32)
        o_ref[0] = out.astype(jnp.bfloat16)

    return pl.pallas_call(
        body,
        out_shape=jax.ShapeDtypeStruct((N, 3136, 32), jnp.bfloat16),
        grid=(N,),
        in_specs=[pl.BlockSpec((1, 56, 56, 96), lambda n: (n, 0, 0, 0))]
        + [_full_spec(a) for a in wargs],
        out_specs=pl.BlockSpec((1, 3136, 32), lambda n: (n, 0, 0)),
        compiler_params=_PAR,
    )(x, *wargs)


def _head(h, head_w, head_b, cls_w, cls_b):
    N = h.shape[0]
    HC = head_w.shape[1]
    NC = cls_w.shape[1]
    G = 2
    rows = (N // G) * 49
    flat = h.reshape(N * 49, 448)

    def body(x_ref, hw, hb, cw, cb, o_ref):
        y = jnp.dot(x_ref[...], hw[...],
                    preferred_element_type=jnp.float32) + hb[...]
        y = y * jax.nn.sigmoid(y)
        pooled = jnp.mean(y.reshape(N // G, 49, HC), axis=1)
        logits = jnp.dot(pooled.astype(jnp.bfloat16), cw[...],
                         preferred_element_type=jnp.float32) + cb[...]
        o_ref[...] = logits

    return pl.pallas_call(
        body,
        out_shape=jax.ShapeDtypeStruct((N, NC), jnp.float32),
        grid=(G,),
        in_specs=[pl.BlockSpec((rows, 448), lambda i: (i, 0)),
                  _full_spec(head_w), _full_spec(head_b),
                  _full_spec(cls_w), _full_spec(cls_b)],
        out_specs=pl.BlockSpec((N // G, NC), lambda i: (i, 0)),
        compiler_params=_PAR,
    )(flat, head_w, head_b, cls_w, cls_b)


def kernel(x, stem_w, stem_b, block0_dw_w, block0_dw_b, block0_se_w1, block0_se_b1, block0_se_w2, block0_se_b2, block0_proj_w, block0_proj_b, block1_exp_w, block1_exp_b, block1_dw_w, block1_dw_b, block1_se_w1, block1_se_b1, block1_se_w2, block1_se_b2, block1_proj_w, block1_proj_b, block2_exp_w, block2_exp_b, block2_dw_w, block2_dw_b, block2_se_w1, block2_se_b1, block2_se_w2, block2_se_b2, block2_proj_w, block2_proj_b, block3_exp_w, block3_exp_b, block3_dw_w, block3_dw_b, block3_se_w1, block3_se_b1, block3_se_w2, block3_se_b2, block3_proj_w, block3_proj_b, block4_exp_w, block4_exp_b, block4_dw_w, block4_dw_b, block4_se_w1, block4_se_b1, block4_se_w2, block4_se_b2, block4_proj_w, block4_proj_b, block5_exp_w, block5_exp_b, block5_dw_w, block5_dw_b, block5_se_w1, block5_se_b1, block5_se_w2, block5_se_b2, block5_proj_w, block5_proj_b, block6_exp_w, block6_exp_b, block6_dw_w, block6_dw_b, block6_se_w1, block6_se_b1, block6_se_w2, block6_se_b2, block6_proj_w, block6_proj_b, block7_exp_w, block7_exp_b, block7_dw_w, block7_dw_b, block7_se_w1, block7_se_b1, block7_se_w2, block7_se_b2, block7_proj_w, block7_proj_b, head_w, head_b, cls_w, cls_b):
    blocks = [
        {"dw_w": block0_dw_w, "dw_b": block0_dw_b,
         "se_w1": block0_se_w1, "se_b1": block0_se_b1,
         "se_w2": block0_se_w2, "se_b2": block0_se_b2,
         "proj_w": block0_proj_w, "proj_b": block0_proj_b},
        {"exp_w": block1_exp_w, "exp_b": block1_exp_b,
         "dw_w": block1_dw_w, "dw_b": block1_dw_b,
         "se_w1": block1_se_w1, "se_b1": block1_se_b1,
         "se_w2": block1_se_w2, "se_b2": block1_se_b2,
         "proj_w": block1_proj_w, "proj_b": block1_proj_b},
        {"exp_w": block2_exp_w, "exp_b": block2_exp_b,
         "dw_w": block2_dw_w, "dw_b": block2_dw_b,
         "se_w1": block2_se_w1, "se_b1": block2_se_b1,
         "se_w2": block2_se_w2, "se_b2": block2_se_b2,
         "proj_w": block2_proj_w, "proj_b": block2_proj_b},
        {"exp_w": block3_exp_w, "exp_b": block3_exp_b,
         "dw_w": block3_dw_w, "dw_b": block3_dw_b,
         "se_w1": block3_se_w1, "se_b1": block3_se_b1,
         "se_w2": block3_se_w2, "se_b2": block3_se_b2,
         "proj_w": block3_proj_w, "proj_b": block3_proj_b},
        {"exp_w": block4_exp_w, "exp_b": block4_exp_b,
         "dw_w": block4_dw_w, "dw_b": block4_dw_b,
         "se_w1": block4_se_w1, "se_b1": block4_se_b1,
         "se_w2": block4_se_w2, "se_b2": block4_se_b2,
         "proj_w": block4_proj_w, "proj_b": block4_proj_b},
        {"exp_w": block5_exp_w, "exp_b": block5_exp_b,
         "dw_w": block5_dw_w, "dw_b": block5_dw_b,
         "se_w1": block5_se_w1, "se_b1": block5_se_b1,
         "se_w2": block5_se_w2, "se_b2": block5_se_b2,
         "proj_w": block5_proj_w, "proj_b": block5_proj_b},
        {"exp_w": block6_exp_w, "exp_b": block6_exp_b,
         "dw_w": block6_dw_w, "dw_b": block6_dw_b,
         "se_w1": block6_se_w1, "se_b1": block6_se_b1,
         "se_w2": block6_se_w2, "se_b2": block6_se_b2,
         "proj_w": block6_proj_w, "proj_b": block6_proj_b},
        {"exp_w": block7_exp_w, "exp_b": block7_exp_b,
         "dw_w": block7_dw_w, "dw_b": block7_dw_b,
         "se_w1": block7_se_w1, "se_b1": block7_se_b1,
         "se_w2": block7_se_w2, "se_b2": block7_se_b2,
         "proj_w": block7_proj_w, "proj_b": block7_proj_b},
    ]
    h = _stem_b0(x, stem_w, stem_b, blocks[0])
    h = _b1_b2(h, blocks[1], blocks[2])
    H = W = 56
    for bi, (cfg, p) in enumerate(zip(_CFGS, blocks)):
        if bi < 3:
            continue
        if cfg[3] == 2:
            h = _s2d(h, H, W)
        h, H, W = _mbconv(h, H, W, cfg, p)
    return _head(h.reshape(x.shape[0], H * W, _CFGS[-1][1]),
                 head_w, head_b, cls_w, cls_b)


# speedup vs baseline: 1.0131x; 1.0131x over previous
import jax
import jax.numpy as jnp
from jax import lax
from jax.experimental import pallas as pl
from jax.experimental.pallas import tpu as pltpu

_PAR = pltpu.CompilerParams(dimension_semantics=("parallel",))

_CFGS = [
    (48, 24, 3, 1, 1),
    (24, 32, 3, 2, 6),
    (32, 32, 3, 1, 6),
    (32, 56, 5, 2, 6),
    (56, 112, 3, 2, 6),
    (112, 160, 5, 1, 6),
    (160, 272, 5, 2, 6),
    (272, 448, 3, 1, 6),
]


def _same_pad(size, k, s):
    out = -(-size // s)
    pad = max((out - 1) * s + k - size, 0)
    return pad // 2, pad - pad // 2, out


def _full_spec(a):
    nd = a.ndim
    return pl.BlockSpec(a.shape, lambda n, _nd=nd: (0,) * _nd)


def _s2d(x, H, W):
    N, _, C = x.shape
    y = x.reshape(N, H // 2, 2, W // 2, 2, C)
    y = jnp.transpose(y, (0, 1, 3, 2, 4, 5))
    return y.reshape(N, H // 2, W // 2, 4 * C)


def _mbconv(x, H, W, cfg, p):
    in_ch, out_ch, k, s, expand = cfg
    N = x.shape[0]
    mid = in_ch * expand
    pt, pb, Ho = _same_pad(H, k, s)
    pL, pR, Wo = _same_pad(W, k, s)
    use_res = (s == 1 and in_ch == out_ch)
    has_exp = expand != 1

    if s == 2:
        H2, W2 = H // 2, W // 2
        TA = max(0, -min((kh - pt) // 2 for kh in range(k)))
        BA = max(0, max((kh - pt) // 2 for kh in range(k)) + Ho - H2)
        LA = max(0, -min((kw - pL) // 2 for kw in range(k)))
        RA = max(0, max((kw - pL) // 2 for kw in range(k)) + Wo - W2)
        x_in = x
        in_spec = pl.BlockSpec((1, H2, W2, 4 * in_ch),
                               lambda n: (n, 0, 0, 0))
        ew_arg = jnp.kron(jnp.eye(4, dtype=p["exp_w"].dtype), p["exp_w"])
        eb_arg = jnp.tile(p["exp_b"], (1, 4))
    else:
        x_in = x
        in_spec = pl.BlockSpec((1, H * W, in_ch), lambda n: (n, 0, 0))
        if has_exp:
            ew_arg, eb_arg = p["exp_w"], p["exp_b"]

    def body(*refs):
        if has_exp:
            (x_ref, ew, eb, dw, db, s1, c1, s2, c2, pw, pbias, o_ref) = refs
        else:
            (x_ref, dw, db, s1, c1, s2, c2, pw, pbias, o_ref) = refs

        def expand_fn(r):
            if not has_exp:
                return r
            h = jnp.dot(r, ew[...], preferred_element_type=jnp.float32)
            h = h + eb[...]
            return (h * jax.nn.sigmoid(h)).astype(jnp.bfloat16)

        if s == 1:
            x2 = x_ref[0]
            h = expand_fn(x2)
            hp = jnp.pad(h.reshape(H, W, mid),
                         ((pt, pb), (pL, pR), (0, 0)))
            taps = {(0, 0): hp}
        else:
            blk = x_ref[0]
            e = expand_fn(blk.reshape(H2 * W2, 4 * in_ch))
            e3 = e.reshape(H2, W2, 4 * mid)
            taps = {}
            for a in range(2):
                for b2 in range(2):
                    g = (a * 2 + b2) * mid
                    pl_ = e3[:, :, g:g + mid]
                    taps[(a, b2)] = jnp.pad(
                        pl_, ((TA, BA), (LA, RA), (0, 0)))
        acc = jnp.zeros((Ho, Wo, mid), jnp.float32)
        for kh in range(k):
            for kw in range(k):
                if s == 1:
                    src = taps[(0, 0)]
                    ah, aw = kh, kw
                else:
                    src = taps[((kh - pt) % 2, (kw - pL) % 2)]
                    ah = (kh - pt) // 2 + TA
                    aw = (kw - pL) // 2 + LA
                tap = src[ah:ah + Ho, aw:aw + Wo, :]
                acc = acc + tap.astype(jnp.float32) * dw[kh * k + kw]
        y = acc + db[...]
        y = y * jax.nn.sigmoid(y)
        y2 = y.reshape(Ho * Wo, mid)
        pooled = jnp.mean(y2, axis=0, keepdims=True)
        t = jnp.dot(pooled.astype(jnp.bfloat16), s1[...],
                    preferred_element_type=jnp.float32) + c1[...]
        t = t * jax.nn.sigmoid(t)
        g = jnp.dot(t.astype(jnp.bfloat16), s2[...],
                    preferred_element_type=jnp.float32) + c2[...]
        gate = jax.nn.sigmoid(g)
        ys = (y2 * gate).astype(jnp.bfloat16)
        out = jnp.dot(ys, pw[...],
                      preferred_element_type=jnp.float32) + pbias[...]
        if use_res:
            out = out + x2.astype(jnp.float32)
        o_ref[0] = out.astype(jnp.bfloat16)

    names = ["dw_w", "dw_b", "se_w1", "se_b1", "se_w2", "se_b2",
             "proj_w", "proj_b"]
    args = [x_in] + ([ew_arg, eb_arg] if has_exp else []) \
        + [p[nm] for nm in names]
    specs = [in_spec]
    specs += [_full_spec(a) for a in args[1:]]
    out = pl.pallas_call(
        body,
        out_shape=jax.ShapeDtypeStruct((N, Ho * Wo, out_ch), jnp.bfloat16),
        grid=(N,),
        in_specs=specs,
        out_specs=pl.BlockSpec((1, Ho * Wo, out_ch), lambda n: (n, 0, 0)),
        compiler_params=_PAR,
    )(*args)
    return out, Ho, Wo


def _stem_b0(x, w, b, p):
    N = x.shape[0]
    C = 48
    xs = x.astype(jnp.bfloat16).reshape(N, 3, 56, 4, 56, 4)
    xs = jnp.transpose(xs, (0, 2, 4, 3, 5, 1)).reshape(N, 56, 56, 48)
    pw4 = jnp.kron(jnp.eye(4, dtype=p["proj_w"].dtype), p["proj_w"])
    pb4 = jnp.tile(p["proj_b"], (1, 4))

    def body(x_ref, w_ref, b_ref, dw, db, s1, c1, s2, c2, pwr, pbr, o_ref):
        src = jnp.pad(x_ref[0], ((0, 1), (0, 1), (0, 0)))
        planes = {}
        for u in range(2):
            for v in range(2):
                pieces = []
                for kh in range(3):
                    for kw in range(3):
                        ish, a = (2 * u + kh) // 4, (2 * u + kh) % 4
                        jsh, bb = (2 * v + kw) // 4, (2 * v + kw) % 4
                        lb = (a * 4 + bb) * 3
                        pieces.append(src[ish:ish + 56, jsh:jsh + 56,
                                          lb:lb + 3])
                pat = jnp.concatenate(pieces, axis=-1).reshape(56 * 56, 27)
                yv = jnp.dot(pat, w_ref[...],
                             preferred_element_type=jnp.float32) + b_ref[...]
                yv = yv * jax.nn.sigmoid(yv)
                st = yv.astype(jnp.bfloat16).reshape(56, 56, 48)
                planes[(u, v)] = jnp.pad(st, ((1, 1), (1, 1), (0, 0)))
        ys = []
        pooled = None
        for u in range(2):
            for v in range(2):
                acc = jnp.zeros((56, 56, C), jnp.float32)
                for kh in range(3):
                    for kw in range(3):
                        ra, d = (u + kh - 1) % 2, (u + kh - 1) // 2 + 1
                        ca, e = (v + kw - 1) % 2, (v + kw - 1) // 2 + 1
                        tap = planes[(ra, ca)][d:d + 56, e:e + 56, :]
                        acc = acc + tap.astype(jnp.float32) * dw[kh * 3 + kw]
                yv = acc + db[...]
                yv = yv * jax.nn.sigmoid(yv)
                ys.append(yv)
                pm = jnp.mean(yv.reshape(56 * 56, C), axis=0, keepdims=True)
                pooled = pm if pooled is None else pooled + pm
        pooled = pooled * 0.25
        t = jnp.dot(pooled.astype(jnp.bfloat16), s1[...],
                    preferred_element_type=jnp.float32) + c1[...]
        t = t * jax.nn.sigmoid(t)
        g = jnp.dot(t.astype(jnp.bfloat16), s2[...],
                    preferred_element_type=jnp.float32) + c2[...]
        gate = jax.nn.sigmoid(g)
        gated = [(y.reshape(56 * 56, C) * gate).astype(jnp.bfloat16)
                 for y in ys]
        big = jnp.concatenate(gated, axis=-1)
        out = jnp.dot(big, pwr[...],
                      preferred_element_type=jnp.float32) + pbr[...]
        o_ref[0] = out.astype(jnp.bfloat16).reshape(56, 56, 96)

    wargs = [w, b, p["dw_w"], p["dw_b"], p["se_w1"], p["se_b1"],
             p["se_w2"], p["se_b2"], pw4, pb4]
    return pl.pallas_call(
        body,
        out_shape=jax.ShapeDtypeStruct((N, 56, 56, 96), jnp.bfloat16),
        grid=(N,),
        in_specs=[pl.BlockSpec((1, 56, 56, 48), lambda n: (n, 0, 0, 0))]
        + [_full_spec(a) for a in wargs],
        out_specs=pl.BlockSpec((1, 56, 56, 96), lambda n: (n, 0, 0, 0)),
        compiler_params=_PAR,
    )(xs, *wargs)


def _b1_b2(x, p1, p2):
    N = x.shape[0]
    m1, m2 = 144, 192
    ew4 = jnp.kron(jnp.eye(4, dtype=p1["exp_w"].dtype), p1["exp_w"])
    eb4 = jnp.tile(p1["exp_b"], (1, 4))
    wargs = [ew4, eb4, p1["dw_w"], p1["dw_b"], p1["se_w1"], p1["se_b1"],
             p1["se_w2"], p1["se_b2"], p1["proj_w"], p1["proj_b"],
             p2["exp_w"], p2["exp_b"], p2["dw_w"], p2["dw_b"], p2["se_w1"],
             p2["se_b1"], p2["se_w2"], p2["se_b2"], p2["proj_w"],
             p2["proj_b"]]

    def body(x_ref, e1w, e1b, d1w, d1b, s11, c11, s12, c12, p1w, p1b,
             e2w, e2b, d2w, d2b, s21, c21, s22, c22, p2w, p2b, o_ref):
        blk = x_ref[0]
        e = jnp.dot(blk.reshape(56 * 56, 96), e1w[...],
                    preferred_element_type=jnp.float32) + e1b[...]
        e = (e * jax.nn.sigmoid(e)).astype(jnp.bfloat16)
        e = e.reshape(56, 56, 4 * m1)
        taps = {}
        for a in range(2):
            for b2_ in range(2):
                g = (a * 2 + b2_) * m1
                pl_ = e[:, :, g:g + m1]
                taps[(a, b2_)] = jnp.pad(pl_, ((0, 1), (0, 1), (0, 0)))
        acc = jnp.zeros((56, 56, m1), jnp.float32)
        for kh in range(3):
            for kw in range(3):
                src = taps[(kh % 2, kw % 2)]
                tap = src[kh // 2:kh // 2 + 56, kw // 2:kw // 2 + 56, :]
                acc = acc + tap.astype(jnp.float32) * d1w[kh * 3 + kw]
        y = acc + d1b[...]
        y = y * jax.nn.sigmoid(y)
        y2 = y.reshape(3136, m1)
        pooled = jnp.mean(y2, axis=0, keepdims=True)
        t = jnp.dot(pooled.astype(jnp.bfloat16), s11[...],
                    preferred_element_type=jnp.float32) + c11[...]
        t = t * jax.nn.sigmoid(t)
        g = jnp.dot(t.astype(jnp.bfloat16), s12[...],
                    preferred_element_type=jnp.float32) + c12[...]
        gate = jax.nn.sigmoid(g)
        ys = (y2 * gate).astype(jnp.bfloat16)
        h1 = jnp.dot(ys, p1w[...],
                     preferred_element_type=jnp.float32) + p1b[...]
        h1 = h1.astype(jnp.bfloat16)
        e2 = jnp.dot(h1, e2w[...],
                     preferred_element_type=jnp.float32) + e2b[...]
        e2 = (e2 * jax.nn.sigmoid(e2)).astype(jnp.bfloat16)
        hp = jnp.pad(e2.reshape(56, 56, m2), ((1, 1), (1, 1), (0, 0)))
        acc2 = jnp.zeros((56, 56, m2), jnp.float32)
        for kh in range(3):
            for kw in range(3):
                tap = hp[kh:kh + 56, kw:kw + 56, :]
                acc2 = acc2 + tap.astype(jnp.float32) * d2w[kh * 3 + kw]
        y = acc2 + d2b[...]
        y = y * jax.nn.sigmoid(y)
        y2 = y.reshape(3136, m2)
        pooled = jnp.mean(y2, axis=0, keepdims=True)
        t = jnp.dot(pooled.astype(jnp.bfloat16), s21[...],
                    preferred_element_type=jnp.float32) + c21[...]
        t = t * jax.nn.sigmoid(t)
        g = jnp.dot(t.astype(jnp.bfloat16), s22[...],
                    preferred_element_type=jnp.float32) + c22[...]
        gate = jax.nn.sigmoid(g)
        ys = (y2 * gate).astype(jnp.bfloat16)
        out = jnp.dot(ys, p2w[...],
                      preferred_element_type=jnp.float32) + p2b[...]
        out = out + h1.astype(jnp.float32)
        o_ref[0] = out.astype(jnp.bfloat16)

    return pl.pallas_call(
        body,
        out_shape=jax.ShapeDtypeStruct((N, 3136, 32), jnp.bfloat16),
        grid=(N,),
        in_specs=[pl.BlockSpec((1, 56, 56, 96), lambda n: (n, 0, 0, 0))]
        + [_full_spec(a) for a in wargs],
        out_specs=pl.BlockSpec((1, 3136, 32), lambda n: (n, 0, 0)),
        compiler_params=_PAR,
    )(x, *wargs)


def _head(h, head_w, head_b, cls_w, cls_b):
    N = h.shape[0]
    HC = head_w.shape[1]
    NC = cls_w.shape[1]
    G = 2
    rows = (N // G) * 49
    flat = h.reshape(N * 49, 448)

    def body(x_ref, hw, hb, cw, cb, o_ref):
        y = jnp.dot(x_ref[...], hw[...],
                    preferred_element_type=jnp.float32) + hb[...]
        y = y * jax.nn.sigmoid(y)
        pooled = jnp.mean(y.reshape(N // G, 49, HC), axis=1)
        logits = jnp.dot(pooled.astype(jnp.bfloat16), cw[...],
                         preferred_element_type=jnp.float32) + cb[...]
        o_ref[...] = logits

    return pl.pallas_call(
        body,
        out_shape=jax.ShapeDtypeStruct((N, NC), jnp.float32),
        grid=(G,),
        in_specs=[pl.BlockSpec((rows, 448), lambda i: (i, 0)),
                  _full_spec(head_w), _full_spec(head_b),
                  _full_spec(cls_w), _full_spec(cls_b)],
        out_specs=pl.BlockSpec((N // G, NC), lambda i: (i, 0)),
        compiler_params=_PAR,
    )(flat, head_w, head_b, cls_w, cls_b)


def kernel(x, stem_w, stem_b, block0_dw_w, block0_dw_b, block0_se_w1, block0_se_b1, block0_se_w2, block0_se_b2, block0_proj_w, block0_proj_b, block1_exp_w, block1_exp_b, block1_dw_w, block1_dw_b, block1_se_w1, block1_se_b1, block1_se_w2, block1_se_b2, block1_proj_w, block1_proj_b, block2_exp_w, block2_exp_b, block2_dw_w, block2_dw_b, block2_se_w1, block2_se_b1, block2_se_w2, block2_se_b2, block2_proj_w, block2_proj_b, block3_exp_w, block3_exp_b, block3_dw_w, block3_dw_b, block3_se_w1, block3_se_b1, block3_se_w2, block3_se_b2, block3_proj_w, block3_proj_b, block4_exp_w, block4_exp_b, block4_dw_w, block4_dw_b, block4_se_w1, block4_se_b1, block4_se_w2, block4_se_b2, block4_proj_w, block4_proj_b, block5_exp_w, block5_exp_b, block5_dw_w, block5_dw_b, block5_se_w1, block5_se_b1, block5_se_w2, block5_se_b2, block5_proj_w, block5_proj_b, block6_exp_w, block6_exp_b, block6_dw_w, block6_dw_b, block6_se_w1, block6_se_b1, block6_se_w2, block6_se_b2, block6_proj_w, block6_proj_b, block7_exp_w, block7_exp_b, block7_dw_w, block7_dw_b, block7_se_w1, block7_se_b1, block7_se_w2, block7_se_b2, block7_proj_w, block7_proj_b, head_w, head_b, cls_w, cls_b):
    blocks = [
        {"dw_w": block0_dw_w, "dw_b": block0_dw_b,
         "se_w1": block0_se_w1, "se_b1": block0_se_b1,
         "se_w2": block0_se_w2, "se_b2": block0_se_b2,
         "proj_w": block0_proj_w, "proj_b": block0_proj_b},
        {"exp_w": block1_exp_w, "exp_b": block1_exp_b,
         "dw_w": block1_dw_w, "dw_b": block1_dw_b,
         "se_w1": block1_se_w1, "se_b1": block1_se_b1,
         "se_w2": block1_se_w2, "se_b2": block1_se_b2,
         "proj_w": block1_proj_w, "proj_b": block1_proj_b},
        {"exp_w": block2_exp_w, "exp_b": block2_exp_b,
         "dw_w": block2_dw_w, "dw_b": block2_dw_b,
         "se_w1": block2_se_w1, "se_b1": block2_se_b1,
         "se_w2": block2_se_w2, "se_b2": block2_se_b2,
         "proj_w": block2_proj_w, "proj_b": block2_proj_b},
        {"exp_w": block3_exp_w, "exp_b": block3_exp_b,
         "dw_w": block3_dw_w, "dw_b": block3_dw_b,
         "se_w1": block3_se_w1, "se_b1": block3_se_b1,
         "se_w2": block3_se_w2, "se_b2": block3_se_b2,
         "proj_w": block3_proj_w, "proj_b": block3_proj_b},
        {"exp_w": block4_exp_w, "exp_b": block4_exp_b,
         "dw_w": block4_dw_w, "dw_b": block4_dw_b,
         "se_w1": block4_se_w1, "se_b1": block4_se_b1,
         "se_w2": block4_se_w2, "se_b2": block4_se_b2,
         "proj_w": block4_proj_w, "proj_b": block4_proj_b},
        {"exp_w": block5_exp_w, "exp_b": block5_exp_b,
         "dw_w": block5_dw_w, "dw_b": block5_dw_b,
         "se_w1": block5_se_w1, "se_b1": block5_se_b1,
         "se_w2": block5_se_w2, "se_b2": block5_se_b2,
         "proj_w": block5_proj_w, "proj_b": block5_proj_b},
        {"exp_w": block6_exp_w, "exp_b": block6_exp_b,
         "dw_w": block6_dw_w, "dw_b": block6_dw_b,
         "se_w1": block6_se_w1, "se_b1": block6_se_b1,
         "se_w2": block6_se_w2, "se_b2": block6_se_b2,
         "proj_w": block6_proj_w, "proj_b": block6_proj_b},
        {"exp_w": block7_exp_w, "exp_b": block7_exp_b,
         "dw_w": block7_dw_w, "dw_b": block7_dw_b,
         "se_w1": block7_se_w1, "se_b1": block7_se_b1,
         "se_w2": block7_se_w2, "se_b2": block7_se_b2,
         "proj_w": block7_proj_w, "proj_b": block7_proj_b},
    ]
    h = _stem_b0(x, stem_w, stem_b, blocks[0])
    h = _b1_b2(h, blocks[1], blocks[2])
    H = W = 56
    for bi, (cfg, p) in enumerate(zip(_CFGS, blocks)):
        if bi < 3:
            continue
        if cfg[3] == 2:
            h = _s2d(h, H, W)
        h, H, W = _mbconv(h, H, W, cfg, p)
    return _head(h.reshape(x.shape[0], H * W, _CFGS[-1][1]),
                 head_w, head_b, cls_w, cls_b)
